# speedup vs baseline: 3.0447x; 3.0447x over previous
import jax
import jax.numpy as jnp
from jax.experimental import pallas as pl
from jax.experimental.pallas import tpu as pltpu


def _copy_kernel(idx_ref, in_ref, out_ref):
    out_ref[...] = in_ref[...]


def kernel(input, indices):
    B, C, H, W = input.shape
    grid_spec = pltpu.PrefetchScalarGridSpec(
        num_scalar_prefetch=1,
        grid=(C,),
        in_specs=[
            pl.BlockSpec((B, 1, H, W), lambda c, idx: (0, idx[c], 0, 0)),
        ],
        out_specs=pl.BlockSpec((B, 1, H, W), lambda c, idx: (0, c, 0, 0)),
    )
    return pl.pallas_call(
        _copy_kernel,
        grid_spec=grid_spec,
        out_shape=jax.ShapeDtypeStruct(input.shape, input.dtype),
    )(indices, input)
